# baseline (speedup 1.0000x reference)
import jax
import jax.numpy as jnp
from jax import lax
from jax.experimental import pallas as pl
from jax.experimental.pallas import tpu as pltpu
from jax.experimental.pallas import tpu_sc as plsc

B, N, S, D = 8, 14, 196, 128
ROW = S * D
NCHUNK = 4
CHUNK = ROW // NCHUNK
NGRP = 49


def _body(x_hbm, out_hbm, buf, sem_in, sem_out):
    c = lax.axis_index("c")
    s = lax.axis_index("s")
    wid = s * 2 + c
    b = wid // NCHUNK
    ch = lax.rem(wid, NCHUNK)

    for n in range(N):
        pltpu.async_copy(x_hbm.at[b, n, ch], buf.at[n], sem_in)
    for n in range(N):
        pltpu.make_async_copy(x_hbm.at[b, n, ch], buf.at[n], sem_in).wait()

    def fire(g, carry):
        for p in range(6):
            n_src = jnp.where(
                (g > 0) & (lax.rem(g - 1, 6) == p),
                lax.div(g - 1, 6),
                8 + p,
            )
            pltpu.async_copy(buf.at[n_src], out_hbm.at[b, g * 6 + p, ch], sem_out)
        return carry

    lax.fori_loop(0, NGRP, fire, 0)

    def drain(g, carry):
        for p in range(6):
            pltpu.make_async_copy(
                buf.at[0], out_hbm.at[b, g * 6 + p, ch], sem_out
            ).wait()
        return carry

    lax.fori_loop(0, NGRP, drain, 0)


@jax.jit
def kernel(x):
    x4 = x.reshape(B, N, NCHUNK, CHUNK)
    out = pl.kernel(
        _body,
        out_type=jax.ShapeDtypeStruct((B, 6 * NGRP, NCHUNK, CHUNK), jnp.float32),
        mesh=plsc.VectorSubcoreMesh(core_axis_name="c", subcore_axis_name="s"),
        scratch_types=[
            pltpu.VMEM((N, CHUNK), jnp.float32),
            pltpu.SemaphoreType.DMA,
            pltpu.SemaphoreType.DMA,
        ],
    )(x4)
    return out.reshape(B, 6, NGRP, S, D)


# speedup vs baseline: 2.3950x; 2.3950x over previous
import jax
import jax.numpy as jnp
from jax import lax
from jax.experimental import pallas as pl
from jax.experimental.pallas import tpu as pltpu
from jax.experimental.pallas import tpu_sc as plsc

B, N, S, D = 8, 14, 196, 128
NGRP = 49


def _body(x_hbm, out_hbm, buf, sem_in, sem_out):
    c = lax.axis_index("c")
    s = lax.axis_index("s")
    wid = s * 2 + c
    b = wid // 4
    role = lax.rem(wid, 4)

    def out_at(j):
        return out_hbm.at[b, j // NGRP, lax.rem(j, NGRP)]

    @pl.when(role < 3)
    def _heavy():
        for pp in range(2):
            pltpu.async_copy(x_hbm.at[b, 8 + 2 * role + pp], buf.at[pp], sem_in)
        for pp in range(2):
            pltpu.make_async_copy(x_hbm.at[b, 0], buf.at[pp], sem_in).wait()

        def fire(g, carry):
            for pp in range(2):
                p = 2 * role + pp
                keep = (g == 0) | (lax.rem(g - 1, 6) != p)

                @pl.when(keep)
                def _():
                    pltpu.async_copy(buf.at[pp], out_at(g * 6 + p), sem_out)

            return carry

        lax.fori_loop(0, NGRP, fire, 0)

        def drain(g, carry):
            for pp in range(2):
                p = 2 * role + pp
                keep = (g == 0) | (lax.rem(g - 1, 6) != p)

                @pl.when(keep)
                def _():
                    pltpu.make_async_copy(buf.at[pp], out_at(0), sem_out).wait()

            return carry

        lax.fori_loop(0, NGRP, drain, 0)

    @pl.when(role == 3)
    def _light():
        def plane(i, carry):
            slot = lax.rem(i, 2)

            @pl.when(i >= 2)
            def _():
                for _jj in range(6):
                    pltpu.make_async_copy(buf.at[0], out_at(0), sem_out).wait()

            pltpu.async_copy(x_hbm.at[b, i], buf.at[slot], sem_in)
            pltpu.make_async_copy(x_hbm.at[b, i], buf.at[slot], sem_in).wait()
            for jj in range(6):
                pltpu.async_copy(buf.at[slot], out_at(36 * i + 6 + 7 * jj), sem_out)
            return carry

        lax.fori_loop(0, 8, plane, 0)
        for _jj in range(12):
            pltpu.make_async_copy(buf.at[0], out_at(0), sem_out).wait()


@jax.jit
def kernel(x):
    out = pl.kernel(
        _body,
        out_type=jax.ShapeDtypeStruct((B, 6, NGRP, S, D), jnp.float32),
        mesh=plsc.VectorSubcoreMesh(core_axis_name="c", subcore_axis_name="s"),
        scratch_types=[
            pltpu.VMEM((2, S, D), jnp.float32),
            pltpu.SemaphoreType.DMA,
            pltpu.SemaphoreType.DMA,
        ],
    )(x)
    return out


# speedup vs baseline: 6.2485x; 2.6090x over previous
import jax
import jax.numpy as jnp
from jax import lax
from jax.experimental import pallas as pl
from jax.experimental.pallas import tpu as pltpu
from jax.experimental.pallas import tpu_sc as plsc

B, N, S, D = 8, 14, 196, 128
NTILES = 28
R = S // NTILES
NGRP = 49


def _body(x_hbm, out_hbm, buf, sem_in, sem_out):
    c = lax.axis_index("c")
    s = lax.axis_index("s")
    wid = s * 2 + c

    @pl.when(wid < NTILES)
    def _():
        lo = wid * R

        for n in range(N):
            pltpu.async_copy(x_hbm.at[n, pl.ds(lo, R)], buf.at[n], sem_in)
        for n in range(N):
            pltpu.make_async_copy(
                x_hbm.at[n, pl.ds(lo, R)], buf.at[n], sem_in
            ).wait()

        def fire(g, carry):
            for p in range(6):
                n_src = jnp.where(
                    (g > 0) & (lax.rem(g - 1, 6) == p),
                    lax.div(g - 1, 6),
                    8 + p,
                )
                j = g * 6 + p
                pltpu.async_copy(
                    buf.at[n_src],
                    out_hbm.at[j // NGRP, lax.rem(j, NGRP), pl.ds(lo, R)],
                    sem_out,
                )
            return carry

        lax.fori_loop(0, NGRP, fire, 0)

        def drain(g, carry):
            for p in range(6):
                pltpu.make_async_copy(
                    buf.at[0],
                    out_hbm.at[0, 0, pl.ds(lo, R)],
                    sem_out,
                ).wait()
            return carry

        lax.fori_loop(0, NGRP, drain, 0)


@jax.jit
def kernel(x):
    x_t = x.transpose(1, 2, 0, 3)
    out_t = pl.kernel(
        _body,
        out_type=jax.ShapeDtypeStruct((6, NGRP, S, B, D), jnp.float32),
        mesh=plsc.VectorSubcoreMesh(core_axis_name="c", subcore_axis_name="s"),
        scratch_types=[
            pltpu.VMEM((N, R, B, D), jnp.float32),
            pltpu.SemaphoreType.DMA,
            pltpu.SemaphoreType.DMA,
        ],
    )(x_t)
    return out_t.transpose(3, 0, 1, 2, 4)


# speedup vs baseline: 6.2540x; 1.0009x over previous
import jax
import jax.numpy as jnp
from jax import lax
from jax.experimental import pallas as pl
from jax.experimental.pallas import tpu as pltpu
from jax.experimental.pallas import tpu_sc as plsc

B, N, S, D = 8, 14, 196, 128
NTILES = 28
R = S // NTILES
NGRP = 49


def _body(x_hbm, out_hbm, buf, sem_in, sem_lt, sem_out):
    c = lax.axis_index("c")
    s = lax.axis_index("s")
    wid = s * 2 + c

    @pl.when(wid < NTILES)
    def _():
        lo = wid * R

        for n in range(8, N):
            pltpu.async_copy(x_hbm.at[n, pl.ds(lo, R)], buf.at[n], sem_in)
        for n in range(8):
            pltpu.async_copy(x_hbm.at[n, pl.ds(lo, R)], buf.at[n], sem_lt)
        pltpu.make_async_copy(
            x_hbm.at[pl.ds(8, 6), pl.ds(lo, R)],
            buf.at[pl.ds(8, 6)],
            sem_in,
        ).wait()

        def fire(g, carry):
            for p in range(6):
                keep = (g == 0) | (lax.rem(g - 1, 6) != p)

                @pl.when(keep)
                def _():
                    j = g * 6 + p
                    pltpu.async_copy(
                        buf.at[8 + p],
                        out_hbm.at[j // NGRP, lax.rem(j, NGRP), pl.ds(lo, R)],
                        sem_out,
                    )

            return carry

        lax.fori_loop(0, NGRP, fire, 0)

        pltpu.make_async_copy(
            x_hbm.at[pl.ds(0, 8), pl.ds(lo, R)],
            buf.at[pl.ds(0, 8)],
            sem_lt,
        ).wait()

        def fire_light(g, carry):
            jj = lax.rem(g - 1, 6)
            j = g * 6 + jj
            pltpu.async_copy(
                buf.at[lax.div(g - 1, 6)],
                out_hbm.at[j // NGRP, lax.rem(j, NGRP), pl.ds(lo, R)],
                sem_out,
            )
            return carry

        lax.fori_loop(1, NGRP, fire_light, 0)

        for _ in range(6):
            pltpu.make_async_copy(
                out_hbm.at[0, pl.ds(0, NGRP), pl.ds(lo, R)],
                out_hbm.at[0, pl.ds(0, NGRP), pl.ds(lo, R)],
                sem_out,
            ).wait()


@jax.jit
def kernel(x):
    x_t = x.transpose(1, 2, 0, 3)
    out_t = pl.kernel(
        _body,
        out_type=jax.ShapeDtypeStruct((6, NGRP, S, B, D), jnp.float32),
        mesh=plsc.VectorSubcoreMesh(core_axis_name="c", subcore_axis_name="s"),
        scratch_types=[
            pltpu.VMEM((N, R, B, D), jnp.float32),
            pltpu.SemaphoreType.DMA,
            pltpu.SemaphoreType.DMA,
            pltpu.SemaphoreType.DMA,
        ],
    )(x_t)
    return out_t.transpose(3, 0, 1, 2, 4)


# speedup vs baseline: 6.2883x; 1.0055x over previous
import jax
import jax.numpy as jnp
from jax import lax
from jax.experimental import pallas as pl
from jax.experimental.pallas import tpu as pltpu
from jax.experimental.pallas import tpu_sc as plsc

B, N, S, D = 8, 14, 196, 128
NW = 32
R = 6
RLO = NW * R
RB = S - RLO
NGRP = 49
NJ = 6 * NGRP


def _body(x_hbm, out_hbm, buf, band, sem_in, sem_lt, sem_bd, sem_out):
    c = lax.axis_index("c")
    s = lax.axis_index("s")
    wid = s * 2 + c
    lo = wid * R

    jlo = (wid * NJ) // NW
    jhi = ((wid + 1) * NJ) // NW
    g0 = jlo // 6
    ghi = (jhi - 1) // 6

    for n in range(8, N):
        pltpu.async_copy(x_hbm.at[n, pl.ds(lo, R)], buf.at[n], sem_in)
    for n in range(8):
        pltpu.async_copy(x_hbm.at[n, pl.ds(lo, R)], buf.at[n], sem_lt)
    for p in range(6):
        pltpu.async_copy(x_hbm.at[8 + p, pl.ds(RLO, RB)], band.at[p], sem_bd)
    for t in range(3):
        gt = g0 + t

        @pl.when((gt >= 1) & (gt <= ghi))
        def _():
            pltpu.async_copy(
                x_hbm.at[lax.div(gt - 1, 6), pl.ds(RLO, RB)],
                band.at[6 + t],
                sem_bd,
            )

    pltpu.make_async_copy(
        x_hbm.at[pl.ds(8, 6), pl.ds(lo, R)], buf.at[pl.ds(8, 6)], sem_in
    ).wait()

    def fire(g, carry):
        for p in range(6):
            keep = (g == 0) | (lax.rem(g - 1, 6) != p)

            @pl.when(keep)
            def _():
                j = g * 6 + p
                pltpu.async_copy(
                    buf.at[8 + p],
                    out_hbm.at[j // NGRP, lax.rem(j, NGRP), pl.ds(lo, R)],
                    sem_out,
                )

        return carry

    lax.fori_loop(0, NGRP, fire, 0)

    for p in range(6):
        pltpu.make_async_copy(
            x_hbm.at[8 + p, pl.ds(RLO, RB)], band.at[p], sem_bd
        ).wait()
    for t in range(3):
        gt = g0 + t

        @pl.when((gt >= 1) & (gt <= ghi))
        def _():
            pltpu.make_async_copy(
                x_hbm.at[0, pl.ds(RLO, RB)], band.at[6 + t], sem_bd
            ).wait()

    def fire_band(j, carry):
        g = lax.div(j, 6)
        p = lax.rem(j, 6)
        replaced = (g > 0) & (lax.rem(g - 1, 6) == p)
        slot = jnp.where(replaced, 6 + (g - g0), p)
        pltpu.async_copy(
            band.at[slot],
            out_hbm.at[lax.div(j, NGRP), lax.rem(j, NGRP), pl.ds(RLO, RB)],
            sem_out,
        )
        return carry

    lax.fori_loop(jlo, jhi, fire_band, 0)

    pltpu.make_async_copy(
        x_hbm.at[pl.ds(0, 8), pl.ds(lo, R)], buf.at[pl.ds(0, 8)], sem_lt
    ).wait()

    def fire_light(g, carry):
        jj = lax.rem(g - 1, 6)
        j = g * 6 + jj
        pltpu.async_copy(
            buf.at[lax.div(g - 1, 6)],
            out_hbm.at[j // NGRP, lax.rem(j, NGRP), pl.ds(lo, R)],
            sem_out,
        )
        return carry

    lax.fori_loop(1, NGRP, fire_light, 0)

    for _ in range(6):
        pltpu.make_async_copy(
            out_hbm.at[0, pl.ds(0, NGRP), pl.ds(lo, R)],
            out_hbm.at[0, pl.ds(0, NGRP), pl.ds(lo, R)],
            sem_out,
        ).wait()

    def drain_band(j, carry):
        pltpu.make_async_copy(
            band.at[0], out_hbm.at[0, 0, pl.ds(RLO, RB)], sem_out
        ).wait()
        return carry

    lax.fori_loop(jlo, jhi, drain_band, 0)


@jax.jit
def kernel(x):
    x_t = x.transpose(1, 2, 0, 3)
    out_t = pl.kernel(
        _body,
        out_type=jax.ShapeDtypeStruct((6, NGRP, S, B, D), jnp.float32),
        mesh=plsc.VectorSubcoreMesh(core_axis_name="c", subcore_axis_name="s"),
        scratch_types=[
            pltpu.VMEM((N, R, B, D), jnp.float32),
            pltpu.VMEM((9, RB, B, D), jnp.float32),
            pltpu.SemaphoreType.DMA,
            pltpu.SemaphoreType.DMA,
            pltpu.SemaphoreType.DMA,
            pltpu.SemaphoreType.DMA,
        ],
    )(x_t)
    return out_t.transpose(3, 0, 1, 2, 4)
